# Initial kernel scaffold; baseline (speedup 1.0000x reference)
import jax
import jax.numpy as jnp
from jax.experimental import pallas as pl


def kernel(x, table):
    raise NotImplementedError("write your pallas kernel here")


# speedup vs baseline: 3.6203x; 3.6203x over previous
import jax
import jax.numpy as jnp
from jax import lax
from jax.experimental import pallas as pl
from jax.experimental.pallas import tpu as pltpu
from jax.experimental.pallas import tpu_sc as plsc

_B = 4
_L = 8192
_D = 1024

_info = plsc.get_sparse_core_info()
_NC = _info.num_cores
_NS = _info.num_subcores
_NW = _NC * _NS
_ROWS_PER_W = _L // _NW
_CHUNK = 64
_NCHUNK = _ROWS_PER_W // _CHUNK


def _copy_body(table_hbm, out_hbm, buf, sem):
    wid = lax.axis_index("s") * _NC + lax.axis_index("c")
    base = wid * _ROWS_PER_W
    for i in range(_NCHUNK):
        row0 = base + i * _CHUNK
        pltpu.async_copy(table_hbm.at[pl.ds(row0, _CHUNK)], buf, sem).wait()
        for b in range(_B):
            pltpu.async_copy(buf, out_hbm.at[b, pl.ds(row0, _CHUNK)], sem).wait()


def kernel(x, table):
    del x
    mesh = plsc.VectorSubcoreMesh(core_axis_name="c", subcore_axis_name="s")
    run = pl.kernel(
        _copy_body,
        mesh=mesh,
        out_type=jax.ShapeDtypeStruct((_B, _L, _D), jnp.float32),
        scratch_types=[
            pltpu.VMEM((_CHUNK, _D), jnp.float32),
            pltpu.SemaphoreType.DMA,
        ],
    )
    return run(table)
